# baseline (speedup 1.0000x reference)
import functools

import jax
import jax.numpy as jnp
from jax import lax
from jax.experimental import pallas as pl
from jax.experimental.pallas import tpu as pltpu
from jax.experimental.pallas import tpu_sc as plsc

HEADS = 4
K_DIM = 512
KNN = 16
N_KEYS = 256
IN_DIM = 2048
OUT_DIM = 1024

TB = 256


def _extract16(s, pay):
    outs, outi = [], []
    for _ in range(16):
        m = jnp.max(s, axis=0, keepdims=True)
        eq = s == m
        am = jnp.max(jnp.where(eq, pay, -1), axis=0, keepdims=True)
        outs.append(m)
        outi.append(255 - am)
        s = jnp.where(eq & (pay == am), -jnp.inf, s)
    return outs, outi


def _topk_tc_kernel(x_ref, wq_ref, bq_ref, k1_ref, k2_ref, idx_ref, w_ref):
    x = x_ref[...]
    q = jnp.dot(x, wq_ref[...], preferred_element_type=jnp.float32) + bq_ref[...]
    sub16 = lax.broadcasted_iota(jnp.int32, (16, TB), 0)
    sub256 = lax.broadcasted_iota(jnp.int32, (256, TB), 0)
    pay256 = 255 - sub256
    idx_parts = []
    w_parts = []
    half = K_DIM // 2
    for h in range(HEADS):
        q1 = q[:, h * K_DIM : h * K_DIM + half]
        q2 = q[:, h * K_DIM + half : (h + 1) * K_DIM]
        s1 = lax.dot_general(k1_ref[h], q1, (((1,), (1,)), ((), ())),
                             preferred_element_type=jnp.float32)
        s2 = lax.dot_general(k2_ref[h], q2, (((1,), (1,)), ((), ())),
                             preferred_element_type=jnp.float32)
        rs1, ki1 = _extract16(s1, pay256)
        rs2, ki2 = _extract16(s2, pay256)
        k1s = jnp.concatenate(ki1, axis=0)
        k2s = jnp.concatenate(ki2, axis=0)
        rs2c = jnp.concatenate(rs2, axis=0)
        cand_f = jnp.concatenate([rs1[i] + rs2c for i in range(16)], axis=0)
        sc_rows, lin_rows = _extract16(cand_f, pay256)
        idx_rows = []
        for lin in lin_rows:
            i_k = lin >> 4
            j_k = lin & 15
            sel1 = sub16 == i_k
            sel2 = sub16 == j_k
            key1 = jnp.max(jnp.where(sel1, k1s, 0), axis=0, keepdims=True)
            key2 = jnp.max(jnp.where(sel2, k2s, 0), axis=0, keepdims=True)
            idx_rows.append(key1 * N_KEYS + key2)
        e = [jnp.exp(s - sc_rows[0]) for s in sc_rows]
        denom = e[0]
        for k in range(1, KNN):
            denom = denom + e[k]
        inv = 1.0 / denom
        w_parts.extend([ek * inv for ek in e])
        idx_parts.extend(idx_rows)
    idx_ref[...] = jnp.concatenate(idx_parts, axis=0)
    w_ref[...] = jnp.concatenate(w_parts, axis=0)


def _route_tc(xf, Wq, bq, keys):
    T = xf.shape[0]
    k1 = keys[:, 0]
    k2 = keys[:, 1]
    grid = T // TB
    idx, w = pl.pallas_call(
        _topk_tc_kernel,
        grid=(grid,),
        in_specs=[
            pl.BlockSpec((TB, IN_DIM), lambda i: (i, 0)),
            pl.BlockSpec((IN_DIM, HEADS * K_DIM), lambda i: (0, 0)),
            pl.BlockSpec((1, HEADS * K_DIM), lambda i: (0, 0)),
            pl.BlockSpec((HEADS, N_KEYS, K_DIM // 2), lambda i: (0, 0, 0)),
            pl.BlockSpec((HEADS, N_KEYS, K_DIM // 2), lambda i: (0, 0, 0)),
        ],
        out_specs=[
            pl.BlockSpec((HEADS * KNN, TB), lambda i: (0, i)),
            pl.BlockSpec((HEADS * KNN, TB), lambda i: (0, i)),
        ],
        out_shape=[
            jax.ShapeDtypeStruct((HEADS * KNN, T), jnp.int32),
            jax.ShapeDtypeStruct((HEADS * KNN, T), jnp.float32),
        ],
    )(xf, Wq, bq.reshape(1, -1), k1, k2)
    return idx, w


def _bag_sc(values, idx, w_exp):
    T = idx.shape[0]
    NW = 32
    tok_per_w = T // NW
    R = HEADS * KNN

    mesh = plsc.VectorSubcoreMesh(core_axis_name="c", subcore_axis_name="s")

    HR = R // 2

    @functools.partial(
        pl.kernel,
        mesh=mesh,
        out_type=jax.ShapeDtypeStruct((T, OUT_DIM), jnp.float32),
        scratch_types=[
            pltpu.VMEM((tok_per_w, R), jnp.int32),
            pltpu.VMEM((R * 16,), jnp.float32),
            pltpu.VMEM((2, HR, OUT_DIM), jnp.float32),
            pltpu.VMEM((OUT_DIM,), jnp.float32),
            pltpu.SemaphoreType.DMA,
            pltpu.SemaphoreType.DMA,
        ],
    )
    def bag(values_hbm, idx_hbm, w_hbm, out_hbm, idx_v, w_v, rows_v, acc_v,
            sem0, sem1):
        wid = lax.axis_index("s") * 2 + lax.axis_index("c")
        base = wid * tok_per_w
        sems = (sem0, sem1)

        pltpu.sync_copy(idx_hbm.at[pl.ds(base, tok_per_w)], idx_v)

        def issue(t, h):
            pltpu.async_copy(
                values_hbm.at[idx_v.at[t, pl.ds(h * HR, HR)]],
                rows_v.at[h], sems[h])

        issue(0, 0)

        def token_body(tl, _):
            pltpu.sync_copy(w_hbm.at[base + tl], w_v)
            for h in range(2):
                buf = h
                if h == 0:
                    issue(tl, 1)
                else:
                    @pl.when(tl < tok_per_w - 1)
                    def _():
                        issue(tl + 1, 0)

                pltpu.make_async_copy(
                    values_hbm.at[idx_v.at[0, pl.ds(0, HR)]],
                    rows_v.at[buf], sems[buf]).wait()
                NACC = 16
                for cg in range(OUT_DIM // (16 * NACC)):
                    if h == 0:
                        init = tuple(jnp.zeros((16,), jnp.float32)
                                     for _ in range(NACC))
                    else:
                        init = tuple(
                            acc_v[pl.ds((cg * NACC + cc) * 16, 16)]
                            for cc in range(NACC))

                    def acc_body(j, accs):
                        wj = w_v[pl.ds((h * HR + j) * 16, 16)]
                        new = []
                        for cc in range(NACC):
                            off = (cg * NACC + cc) * 16
                            new.append(accs[cc]
                                       + wj * rows_v[buf, j, pl.ds(off, 16)])
                        return tuple(new)

                    accs = lax.fori_loop(0, HR, acc_body, init,
                                         unroll=2)
                    for cc in range(NACC):
                        acc_v[pl.ds((cg * NACC + cc) * 16, 16)] = accs[cc]
            pltpu.sync_copy(acc_v, out_hbm.at[base + tl])
            return ()

        lax.fori_loop(0, tok_per_w, token_body, ())

    return bag(values, idx, w_exp)


def kernel(x, Wq, bq, keys, values):
    prefix = x.shape[:-1]
    T = 1
    for d in prefix:
        T *= d
    xf = x.reshape(T, IN_DIM)
    C = 8
    TC_ = T // C
    outs = []
    for c in range(C):
        xc = xf[c * TC_:(c + 1) * TC_]
        idx_t, w_t = _route_tc(xc, Wq, bq, keys)
        idx = idx_t.T
        w_exp = jnp.repeat(w_t.T, 16, axis=1)
        outs.append(_bag_sc(values, idx, w_exp))
    out = jnp.concatenate(outs, axis=0)
    return out.reshape(prefix + (OUT_DIM,))


# speedup vs baseline: 1.0714x; 1.0714x over previous
import functools

import jax
import jax.numpy as jnp
from jax import lax
from jax.experimental import pallas as pl
from jax.experimental.pallas import tpu as pltpu
from jax.experimental.pallas import tpu_sc as plsc

HEADS = 4
K_DIM = 512
KNN = 16
N_KEYS = 256
IN_DIM = 2048
OUT_DIM = 1024

TB = 256


def _extract16(s, pay):
    outs, outi = [], []
    for _ in range(16):
        m = jnp.max(s, axis=0, keepdims=True)
        eq = s == m
        am = jnp.max(jnp.where(eq, pay, -1), axis=0, keepdims=True)
        outs.append(m)
        outi.append(255 - am)
        s = jnp.where(eq & (pay == am), -jnp.inf, s)
    return outs, outi


def _topk_tc_kernel(x_ref, wq_ref, bq_ref, k1_ref, k2_ref, idx_ref, w_ref):
    x = x_ref[...]
    q = jnp.dot(x, wq_ref[...], preferred_element_type=jnp.float32) + bq_ref[...]
    sub16 = lax.broadcasted_iota(jnp.int32, (16, TB), 0)
    sub256 = lax.broadcasted_iota(jnp.int32, (256, TB), 0)
    pay256 = 255 - sub256
    idx_parts = []
    w_parts = []
    half = K_DIM // 2
    for h in range(HEADS):
        q1 = q[:, h * K_DIM : h * K_DIM + half]
        q2 = q[:, h * K_DIM + half : (h + 1) * K_DIM]
        s1 = lax.dot_general(k1_ref[h], q1, (((1,), (1,)), ((), ())),
                             preferred_element_type=jnp.float32)
        s2 = lax.dot_general(k2_ref[h], q2, (((1,), (1,)), ((), ())),
                             preferred_element_type=jnp.float32)
        rs1, ki1 = _extract16(s1, pay256)
        rs2, ki2 = _extract16(s2, pay256)
        k1s = jnp.concatenate(ki1, axis=0)
        k2s = jnp.concatenate(ki2, axis=0)
        rs2c = jnp.concatenate(rs2, axis=0)
        cand_f = jnp.concatenate([rs1[i] + rs2c for i in range(16)], axis=0)
        sc_rows, lin_rows = _extract16(cand_f, pay256)
        idx_rows = []
        for lin in lin_rows:
            i_k = lin >> 4
            j_k = lin & 15
            sel1 = sub16 == i_k
            sel2 = sub16 == j_k
            key1 = jnp.max(jnp.where(sel1, k1s, 0), axis=0, keepdims=True)
            key2 = jnp.max(jnp.where(sel2, k2s, 0), axis=0, keepdims=True)
            idx_rows.append(key1 * N_KEYS + key2)
        e = [jnp.exp(s - sc_rows[0]) for s in sc_rows]
        denom = e[0]
        for k in range(1, KNN):
            denom = denom + e[k]
        inv = 1.0 / denom
        w_parts.extend([ek * inv for ek in e])
        idx_parts.extend(idx_rows)
    idx_ref[...] = jnp.concatenate(idx_parts, axis=0)
    w_ref[...] = jnp.concatenate(w_parts, axis=0)


def _route_tc(xf, Wq, bq, keys):
    T = xf.shape[0]
    k1 = keys[:, 0]
    k2 = keys[:, 1]
    grid = T // TB
    idx, w = pl.pallas_call(
        _topk_tc_kernel,
        grid=(grid,),
        in_specs=[
            pl.BlockSpec((TB, IN_DIM), lambda i: (i, 0)),
            pl.BlockSpec((IN_DIM, HEADS * K_DIM), lambda i: (0, 0)),
            pl.BlockSpec((1, HEADS * K_DIM), lambda i: (0, 0)),
            pl.BlockSpec((HEADS, N_KEYS, K_DIM // 2), lambda i: (0, 0, 0)),
            pl.BlockSpec((HEADS, N_KEYS, K_DIM // 2), lambda i: (0, 0, 0)),
        ],
        out_specs=[
            pl.BlockSpec((HEADS * KNN, TB), lambda i: (0, i)),
            pl.BlockSpec((HEADS * KNN, TB), lambda i: (0, i)),
        ],
        out_shape=[
            jax.ShapeDtypeStruct((HEADS * KNN, T), jnp.int32),
            jax.ShapeDtypeStruct((HEADS * KNN, T), jnp.float32),
        ],
    )(xf, Wq, bq.reshape(1, -1), k1, k2)
    return idx, w


def _bag_sc(values, idx, w_exp):
    T = idx.shape[0]
    NW = 32
    tok_per_w = T // NW
    R = HEADS * KNN

    mesh = plsc.VectorSubcoreMesh(core_axis_name="c", subcore_axis_name="s")

    HR = R // 2

    @functools.partial(
        pl.kernel,
        mesh=mesh,
        out_type=jax.ShapeDtypeStruct((T, OUT_DIM), jnp.float32),
        scratch_types=[
            pltpu.VMEM((tok_per_w, R), jnp.int32),
            pltpu.VMEM((R * 16,), jnp.float32),
            pltpu.VMEM((2, HR, OUT_DIM), jnp.float32),
            pltpu.VMEM((OUT_DIM,), jnp.float32),
            pltpu.SemaphoreType.DMA,
            pltpu.SemaphoreType.DMA,
        ],
    )
    def bag(values_hbm, idx_hbm, w_hbm, out_hbm, idx_v, w_v, rows_v, acc_v,
            sem0, sem1):
        wid = lax.axis_index("s") * 2 + lax.axis_index("c")
        base = wid * tok_per_w
        sems = (sem0, sem1)

        pltpu.sync_copy(idx_hbm.at[pl.ds(base, tok_per_w)], idx_v)

        def issue(t, h):
            pltpu.async_copy(
                values_hbm.at[idx_v.at[t, pl.ds(h * HR, HR)]],
                rows_v.at[h], sems[h])

        issue(0, 0)

        def token_body(tl, _):
            pltpu.sync_copy(w_hbm.at[base + tl], w_v)
            for h in range(2):
                buf = h
                if h == 0:
                    issue(tl, 1)
                else:
                    @pl.when(tl < tok_per_w - 1)
                    def _():
                        issue(tl + 1, 0)

                pltpu.make_async_copy(
                    values_hbm.at[idx_v.at[0, pl.ds(0, HR)]],
                    rows_v.at[buf], sems[buf]).wait()
                NACC = 16
                for cg in range(OUT_DIM // (16 * NACC)):
                    if h == 0:
                        init = tuple(jnp.zeros((16,), jnp.float32)
                                     for _ in range(NACC))
                    else:
                        init = tuple(
                            acc_v[pl.ds((cg * NACC + cc) * 16, 16)]
                            for cc in range(NACC))

                    def acc_body(j, accs):
                        wj = w_v[pl.ds((h * HR + j) * 16, 16)]
                        new = []
                        for cc in range(NACC):
                            off = (cg * NACC + cc) * 16
                            new.append(accs[cc]
                                       + wj * rows_v[buf, j, pl.ds(off, 16)])
                        return tuple(new)

                    accs = lax.fori_loop(0, HR, acc_body, init,
                                         unroll=2)
                    for cc in range(NACC):
                        acc_v[pl.ds((cg * NACC + cc) * 16, 16)] = accs[cc]
            pltpu.sync_copy(acc_v, out_hbm.at[base + tl])
            return ()

        lax.fori_loop(0, tok_per_w, token_body, ())

    return bag(values, idx, w_exp)


def kernel(x, Wq, bq, keys, values):
    prefix = x.shape[:-1]
    T = 1
    for d in prefix:
        T *= d
    xf = x.reshape(T, IN_DIM)
    C = 4
    TC_ = T // C
    outs = []
    for c in range(C):
        xc = xf[c * TC_:(c + 1) * TC_]
        idx_t, w_t = _route_tc(xc, Wq, bq, keys)
        idx = idx_t.T
        w_exp = jnp.repeat(w_t.T, 16, axis=1)
        outs.append(_bag_sc(values, idx, w_exp))
    out = jnp.concatenate(outs, axis=0)
    return out.reshape(prefix + (OUT_DIM,))
